# Initial kernel scaffold; baseline (speedup 1.0000x reference)
import jax
import jax.numpy as jnp
from jax.experimental import pallas as pl


def kernel(x, edge_index, batch, W1, b1, W2, b2):
    raise NotImplementedError("write your pallas kernel here")


# speedup vs baseline: 16.1064x; 16.1064x over previous
import functools

import jax
import jax.numpy as jnp
from jax import lax
from jax.experimental import pallas as pl
from jax.experimental.pallas import tpu as pltpu
from jax.experimental.pallas import tpu_sc as plsc

N = 10000
E = 320000
D = 128
G = 64

NC = 2
NS = 16
NW = NC * NS

EPW = E // NW
CH = 128
NFULL = EPW // CH
TAIL = EPW - NFULL * CH
RPT = N // NS

BN = 1000
NBLK = N // BN

_mesh = plsc.VectorSubcoreMesh(core_axis_name="c", subcore_axis_name="s")
_sc_params = pltpu.CompilerParams(use_tc_tiling_on_sc=False)


@functools.partial(
    pl.kernel,
    out_type=jax.ShapeDtypeStruct((NC, N, 16), jnp.float32),
    mesh=_mesh,
    scratch_types=[
        pltpu.VMEM((CH,), jnp.int32),
        pltpu.VMEM((CH, 16), jnp.float32),
        pltpu.VMEM((TAIL,), jnp.int32),
        pltpu.VMEM_SHARED((N, 16), jnp.float32),
    ],
    compiler_params=_sc_params,
)
def _sc_degree(dst_hbm, ones_hbm, zeros_hbm, out_hbm, idx_v, ones_v, idxt_v,
               acc_sh):
    cid = lax.axis_index("c")
    sid = lax.axis_index("s")
    wid = sid * NC + cid
    rbase = sid * RPT
    pltpu.sync_copy(zeros_hbm.at[pl.ds(rbase, RPT)],
                    acc_sh.at[pl.ds(rbase, RPT)])
    pltpu.sync_copy(ones_hbm, ones_v)
    plsc.subcore_barrier()

    ebase = wid * EPW

    def step(c, carry):
        off = ebase + c * CH
        pltpu.sync_copy(dst_hbm.at[pl.ds(off, CH)], idx_v)
        pltpu.sync_copy(ones_v, acc_sh.at[idx_v], add=True)
        return carry

    lax.fori_loop(0, NFULL, step, 0)
    pltpu.sync_copy(dst_hbm.at[pl.ds(ebase + NFULL * CH, TAIL)], idxt_v)
    pltpu.sync_copy(ones_v.at[pl.ds(0, TAIL)], acc_sh.at[idxt_v], add=True)

    plsc.subcore_barrier()
    pltpu.sync_copy(acc_sh.at[pl.ds(rbase, RPT)],
                    out_hbm.at[cid, pl.ds(rbase, RPT)])


@functools.partial(
    pl.kernel,
    out_type=jax.ShapeDtypeStruct((NC, N, D), jnp.float32),
    mesh=_mesh,
    scratch_types=[
        pltpu.VMEM((CH,), jnp.int32),
        pltpu.VMEM((CH,), jnp.int32),
        pltpu.VMEM((CH, D), jnp.float32),
        pltpu.VMEM((TAIL,), jnp.int32),
        pltpu.VMEM((TAIL,), jnp.int32),
        pltpu.VMEM((TAIL, D), jnp.float32),
        pltpu.VMEM_SHARED((N, D), jnp.float32),
        pltpu.SemaphoreType.DMA,
    ],
    compiler_params=_sc_params,
)
def _sc_edge_sum(y_hbm, src_hbm, dst_hbm, zeros_hbm, out_hbm,
                 idxs_v, idxd_v, rows_v, idxst_v, idxdt_v, rowst_v,
                 acc_sh, sem):
    cid = lax.axis_index("c")
    sid = lax.axis_index("s")
    wid = sid * NC + cid
    rbase = sid * RPT
    pltpu.sync_copy(zeros_hbm.at[pl.ds(rbase, RPT)],
                    acc_sh.at[pl.ds(rbase, RPT)])
    plsc.subcore_barrier()

    ebase = wid * EPW

    def step(c, carry):
        off = ebase + c * CH
        pltpu.sync_copy(src_hbm.at[pl.ds(off, CH)], idxs_v)
        pltpu.sync_copy(dst_hbm.at[pl.ds(off, CH)], idxd_v)
        pltpu.async_copy(y_hbm.at[idxs_v], rows_v, sem).wait()
        pltpu.sync_copy(rows_v, acc_sh.at[idxd_v], add=True)
        return carry

    lax.fori_loop(0, NFULL, step, 0)
    toff = ebase + NFULL * CH
    pltpu.sync_copy(src_hbm.at[pl.ds(toff, TAIL)], idxst_v)
    pltpu.sync_copy(dst_hbm.at[pl.ds(toff, TAIL)], idxdt_v)
    pltpu.async_copy(y_hbm.at[idxst_v], rowst_v, sem).wait()
    pltpu.sync_copy(rowst_v, acc_sh.at[idxdt_v], add=True)

    plsc.subcore_barrier()
    pltpu.sync_copy(acc_sh.at[pl.ds(rbase, RPT)],
                    out_hbm.at[cid, pl.ds(rbase, RPT)])


def _dvec(degp_ref):
    deg = 1.0 + degp_ref[0, :, :1] + degp_ref[1, :, :1]
    return lax.rsqrt(deg)


def _tc1_body(degp_ref, x_ref, w1_ref, y1_ref):
    d = _dvec(degp_ref)
    xw = jnp.dot(x_ref[...], w1_ref[...],
                 preferred_element_type=jnp.float32,
                 precision=lax.Precision.HIGHEST)
    y1_ref[...] = xw * d


def _tc2_body(degp_ref, s_ref, y1_ref, w2_ref, b1_ref, y2_ref):
    d = _dvec(degp_ref)
    s = s_ref[0] + s_ref[1] + y1_ref[...]
    h = jnp.maximum(d * s + b1_ref[...], 0.0)
    hw = jnp.dot(h, w2_ref[...],
                 preferred_element_type=jnp.float32,
                 precision=lax.Precision.HIGHEST)
    y2_ref[...] = hw * d


def _tc3_body(degp_ref, s_ref, y2_ref, b2_ref, batch_ref, out_ref, acc, cnt):
    i = pl.program_id(0)

    @pl.when(i == 0)
    def _init():
        acc[...] = jnp.zeros_like(acc)
        cnt[...] = jnp.zeros_like(cnt)

    d = _dvec(degp_ref)
    o = d * (s_ref[0] + s_ref[1] + y2_ref[...]) + b2_ref[...]
    seg = batch_ref[0, 0, :]
    oh = (lax.broadcasted_iota(jnp.int32, (G, BN), 0)
          == seg[None, :]).astype(jnp.float32)
    acc[...] += jnp.dot(oh, o, preferred_element_type=jnp.float32,
                        precision=lax.Precision.HIGHEST)
    cnt[...] += jnp.sum(oh, axis=1, keepdims=True)

    @pl.when(i == NBLK - 1)
    def _fin():
        out_ref[...] = acc[...] / jnp.maximum(cnt[...], 1.0)


_row = lambda i: (i, 0)
_fix2 = lambda i: (0, 0)

_tc1 = pl.pallas_call(
    _tc1_body,
    grid=(NBLK,),
    in_specs=[
        pl.BlockSpec((NC, BN, 16), lambda i: (0, i, 0)),
        pl.BlockSpec((BN, D), _row),
        pl.BlockSpec((D, D), _fix2),
    ],
    out_specs=pl.BlockSpec((BN, D), _row),
    out_shape=jax.ShapeDtypeStruct((N, D), jnp.float32),
)

_tc2 = pl.pallas_call(
    _tc2_body,
    grid=(NBLK,),
    in_specs=[
        pl.BlockSpec((NC, BN, 16), lambda i: (0, i, 0)),
        pl.BlockSpec((NC, BN, D), lambda i: (0, i, 0)),
        pl.BlockSpec((BN, D), _row),
        pl.BlockSpec((D, D), _fix2),
        pl.BlockSpec((1, D), _fix2),
    ],
    out_specs=pl.BlockSpec((BN, D), _row),
    out_shape=jax.ShapeDtypeStruct((N, D), jnp.float32),
)

_tc3 = pl.pallas_call(
    _tc3_body,
    grid=(NBLK,),
    in_specs=[
        pl.BlockSpec((NC, BN, 16), lambda i: (0, i, 0)),
        pl.BlockSpec((NC, BN, D), lambda i: (0, i, 0)),
        pl.BlockSpec((BN, D), _row),
        pl.BlockSpec((1, D), _fix2),
        pl.BlockSpec((1, 1, BN), lambda i: (i, 0, 0)),
    ],
    out_specs=pl.BlockSpec((G, D), _fix2),
    out_shape=jax.ShapeDtypeStruct((G, D), jnp.float32),
    scratch_shapes=[
        pltpu.VMEM((G, D), jnp.float32),
        pltpu.VMEM((G, 1), jnp.float32),
    ],
)


def kernel(x, edge_index, batch, W1, b1, W2, b2):
    x = x.astype(jnp.float32)
    src = edge_index[0]
    dst = edge_index[1]
    ones16 = jnp.ones((CH, 16), jnp.float32)
    zeros16 = jnp.zeros((N, 16), jnp.float32)
    zerosND = jnp.zeros((N, D), jnp.float32)
    b1r = b1.reshape(1, D)
    b2r = b2.reshape(1, D)
    batch3 = batch.reshape(NBLK, 1, BN)

    degp = _sc_degree(dst, ones16, zeros16)
    y1 = _tc1(degp, x, W1)
    s1 = _sc_edge_sum(y1, src, dst, zerosND)
    y2 = _tc2(degp, s1, y1, W2, b1r)
    s2 = _sc_edge_sum(y2, src, dst, zerosND)
    out = _tc3(degp, s2, y2, b2r, batch3)
    return out


# speedup vs baseline: 24.9435x; 1.5487x over previous
import functools

import jax
import jax.numpy as jnp
from jax import lax
from jax.experimental import pallas as pl
from jax.experimental.pallas import tpu as pltpu
from jax.experimental.pallas import tpu_sc as plsc

N = 10000
E = 320000
D = 128
G = 64

NC = 2
NS = 16
NW = NC * NS

CHR = 100
ROWS = E // CHR
CPT = ROWS // NW
NB = 2
NGRP = CPT // NB
RPT = N // NS

BN = 1000
NBLK = N // BN

_mesh = plsc.VectorSubcoreMesh(core_axis_name="c", subcore_axis_name="s")
_sc_params = pltpu.CompilerParams(use_tc_tiling_on_sc=False)


@functools.partial(
    pl.kernel,
    out_type=jax.ShapeDtypeStruct((NC, N, 16), jnp.float32),
    mesh=_mesh,
    scratch_types=[
        pltpu.VMEM((CPT, CHR), jnp.int32),
        pltpu.VMEM((CHR, 16), jnp.float32),
        pltpu.VMEM_SHARED((N, 16), jnp.float32),
        pltpu.SemaphoreType.DMA,
        pltpu.SemaphoreType.DMA,
    ],
    compiler_params=_sc_params,
)
def _sc_degree(dst_hbm, ones_hbm, zeros_hbm, out_hbm,
               idxd_v, ones_v, acc_sh, s0, s1):
    ssem = (s0, s1)
    cid = lax.axis_index("c")
    sid = lax.axis_index("s")
    wid = sid * NC + cid
    rbase = sid * RPT
    pltpu.sync_copy(zeros_hbm.at[pl.ds(rbase, RPT)],
                    acc_sh.at[pl.ds(rbase, RPT)])
    pltpu.sync_copy(ones_hbm, ones_v)
    pltpu.sync_copy(dst_hbm.at[pl.ds(wid * CPT, CPT)], idxd_v)
    plsc.subcore_barrier()

    def scat(c, b):
        pltpu.async_copy(ones_v, acc_sh.at[idxd_v.at[c]], ssem[b], add=True)

    def swait(c, b):
        pltpu.make_async_copy(ones_v, acc_sh.at[idxd_v.at[c]], ssem[b]).wait()

    def group(j0, carry):
        j = j0 * NB
        for b in range(NB):
            scat(j + b, b)
        for b in range(NB):
            swait(j + b, b)
        return carry

    lax.fori_loop(0, NGRP, group, 0)

    plsc.subcore_barrier()
    pltpu.sync_copy(acc_sh.at[pl.ds(rbase, RPT)],
                    out_hbm.at[cid, pl.ds(rbase, RPT)])


@functools.partial(
    pl.kernel,
    out_type=jax.ShapeDtypeStruct((NC, N, D), jnp.float32),
    mesh=_mesh,
    scratch_types=[
        pltpu.VMEM((CPT, CHR), jnp.int32),
        pltpu.VMEM((CPT, CHR), jnp.int32),
        pltpu.VMEM((NB, CHR, D), jnp.float32),
        pltpu.VMEM_SHARED((N, D), jnp.float32),
        pltpu.SemaphoreType.DMA,
        pltpu.SemaphoreType.DMA,
        pltpu.SemaphoreType.DMA,
        pltpu.SemaphoreType.DMA,
    ],
    compiler_params=_sc_params,
)
def _sc_edge_sum(y_hbm, src_hbm, dst_hbm, zeros_hbm, out_hbm,
                 idxs_v, idxd_v, rows_v, acc_sh,
                 g0, g1, s0, s1):
    gsem = (g0, g1)
    ssem = (s0, s1)
    cid = lax.axis_index("c")
    sid = lax.axis_index("s")
    wid = sid * NC + cid
    rbase = sid * RPT
    pltpu.sync_copy(zeros_hbm.at[pl.ds(rbase, RPT)],
                    acc_sh.at[pl.ds(rbase, RPT)])
    cbase = wid * CPT
    pltpu.sync_copy(src_hbm.at[pl.ds(cbase, CPT)], idxs_v)
    pltpu.sync_copy(dst_hbm.at[pl.ds(cbase, CPT)], idxd_v)
    plsc.subcore_barrier()

    def gath(c, b):
        pltpu.async_copy(y_hbm.at[idxs_v.at[c]], rows_v.at[b], gsem[b])

    def gwait(c, b):
        pltpu.make_async_copy(y_hbm.at[idxs_v.at[c]], rows_v.at[b],
                              gsem[b]).wait()

    def scat(c, b):
        pltpu.async_copy(rows_v.at[b], acc_sh.at[idxd_v.at[c]], ssem[b],
                         add=True)

    def swait(c, b):
        pltpu.make_async_copy(rows_v.at[b], acc_sh.at[idxd_v.at[c]],
                              ssem[b]).wait()

    for b in range(NB):
        gath(b, b)

    def group(j0, carry):
        j = j0 * NB
        for b in range(NB):
            gwait(j + b, b)
            scat(j + b, b)
        for b in range(NB):
            swait(j + b, b)
            gath(j + NB + b, b)
        return carry

    lax.fori_loop(0, NGRP - 1, group, 0)
    j = (NGRP - 1) * NB
    for b in range(NB):
        gwait(j + b, b)
        scat(j + b, b)
    for b in range(NB):
        swait(j + b, b)

    plsc.subcore_barrier()
    pltpu.sync_copy(acc_sh.at[pl.ds(rbase, RPT)],
                    out_hbm.at[cid, pl.ds(rbase, RPT)])


def _dvec(degp_ref):
    deg = 1.0 + degp_ref[0, :, :1] + degp_ref[1, :, :1]
    return lax.rsqrt(deg)


def _tc1_body(degp_ref, x_ref, w1_ref, y1_ref):
    d = _dvec(degp_ref)
    xw = jnp.dot(x_ref[...], w1_ref[...],
                 preferred_element_type=jnp.float32,
                 precision=lax.Precision.HIGHEST)
    y1_ref[...] = xw * d


def _tc2_body(degp_ref, s_ref, y1_ref, w2_ref, b1_ref, y2_ref):
    d = _dvec(degp_ref)
    s = s_ref[0] + s_ref[1] + y1_ref[...]
    h = jnp.maximum(d * s + b1_ref[...], 0.0)
    hw = jnp.dot(h, w2_ref[...],
                 preferred_element_type=jnp.float32,
                 precision=lax.Precision.HIGHEST)
    y2_ref[...] = hw * d


def _tc3_body(degp_ref, s_ref, y2_ref, b2_ref, batch_ref, out_ref, acc, cnt):
    i = pl.program_id(0)

    @pl.when(i == 0)
    def _init():
        acc[...] = jnp.zeros_like(acc)
        cnt[...] = jnp.zeros_like(cnt)

    d = _dvec(degp_ref)
    o = d * (s_ref[0] + s_ref[1] + y2_ref[...]) + b2_ref[...]
    seg = batch_ref[0, 0, :]
    oh = (lax.broadcasted_iota(jnp.int32, (G, BN), 0)
          == seg[None, :]).astype(jnp.float32)
    acc[...] += jnp.dot(oh, o, preferred_element_type=jnp.float32,
                        precision=lax.Precision.HIGHEST)
    cnt[...] += jnp.sum(oh, axis=1, keepdims=True)

    @pl.when(i == NBLK - 1)
    def _fin():
        out_ref[...] = acc[...] / jnp.maximum(cnt[...], 1.0)


_row = lambda i: (i, 0)
_fix2 = lambda i: (0, 0)

_tc1 = pl.pallas_call(
    _tc1_body,
    grid=(NBLK,),
    in_specs=[
        pl.BlockSpec((NC, BN, 16), lambda i: (0, i, 0)),
        pl.BlockSpec((BN, D), _row),
        pl.BlockSpec((D, D), _fix2),
    ],
    out_specs=pl.BlockSpec((BN, D), _row),
    out_shape=jax.ShapeDtypeStruct((N, D), jnp.float32),
)

_tc2 = pl.pallas_call(
    _tc2_body,
    grid=(NBLK,),
    in_specs=[
        pl.BlockSpec((NC, BN, 16), lambda i: (0, i, 0)),
        pl.BlockSpec((NC, BN, D), lambda i: (0, i, 0)),
        pl.BlockSpec((BN, D), _row),
        pl.BlockSpec((D, D), _fix2),
        pl.BlockSpec((1, D), _fix2),
    ],
    out_specs=pl.BlockSpec((BN, D), _row),
    out_shape=jax.ShapeDtypeStruct((N, D), jnp.float32),
)

_tc3 = pl.pallas_call(
    _tc3_body,
    grid=(NBLK,),
    in_specs=[
        pl.BlockSpec((NC, BN, 16), lambda i: (0, i, 0)),
        pl.BlockSpec((NC, BN, D), lambda i: (0, i, 0)),
        pl.BlockSpec((BN, D), _row),
        pl.BlockSpec((1, D), _fix2),
        pl.BlockSpec((1, 1, BN), lambda i: (i, 0, 0)),
    ],
    out_specs=pl.BlockSpec((G, D), _fix2),
    out_shape=jax.ShapeDtypeStruct((G, D), jnp.float32),
    scratch_shapes=[
        pltpu.VMEM((G, D), jnp.float32),
        pltpu.VMEM((G, 1), jnp.float32),
    ],
)


def kernel(x, edge_index, batch, W1, b1, W2, b2):
    x = x.astype(jnp.float32)
    src2d = edge_index[0].reshape(ROWS, CHR)
    dst2d = edge_index[1].reshape(ROWS, CHR)
    ones16 = jnp.ones((CHR, 16), jnp.float32)
    zeros16 = jnp.zeros((N, 16), jnp.float32)
    zerosND = jnp.zeros((N, D), jnp.float32)
    b1r = b1.reshape(1, D)
    b2r = b2.reshape(1, D)
    batch3 = batch.reshape(NBLK, 1, BN)

    degp = _sc_degree(dst2d, ones16, zeros16)
    y1 = _tc1(degp, x, W1)
    s1 = _sc_edge_sum(y1, src2d, dst2d, zerosND)
    y2 = _tc2(degp, s1, y1, W2, b1r)
    s2 = _sc_edge_sum(y2, src2d, dst2d, zerosND)
    out = _tc3(degp, s2, y2, b2r, batch3)
    return out


# speedup vs baseline: 29.3044x; 1.1748x over previous
import functools

import jax
import jax.numpy as jnp
from jax import lax
from jax.experimental import pallas as pl
from jax.experimental.pallas import tpu as pltpu
from jax.experimental.pallas import tpu_sc as plsc

N = 10000
E = 320000
D = 128
G = 64

NC = 2
NS = 16
NW = NC * NS

CHR = 50
ROWS = E // CHR
CPT = ROWS // NW
NB = 4
NGRP = CPT // NB
RPT = N // NS

BN = 1000
NBLK = N // BN

_mesh = plsc.VectorSubcoreMesh(core_axis_name="c", subcore_axis_name="s")
_sc_params = pltpu.CompilerParams(use_tc_tiling_on_sc=False)


@functools.partial(
    pl.kernel,
    out_type=jax.ShapeDtypeStruct((NC, N, 16), jnp.float32),
    mesh=_mesh,
    scratch_types=[
        pltpu.VMEM((CPT, CHR), jnp.int32),
        pltpu.VMEM((CHR, 16), jnp.float32),
        pltpu.VMEM_SHARED((N, 16), jnp.float32),
        pltpu.SemaphoreType.DMA,
        pltpu.SemaphoreType.DMA,
        pltpu.SemaphoreType.DMA,
        pltpu.SemaphoreType.DMA,
    ],
    compiler_params=_sc_params,
)
def _sc_degree(dst_hbm, ones_hbm, zeros_hbm, out_hbm,
               idxd_v, ones_v, acc_sh, s0, s1, s2, s3):
    ssem = (s0, s1, s2, s3)
    cid = lax.axis_index("c")
    sid = lax.axis_index("s")
    wid = sid * NC + cid
    rbase = sid * RPT
    pltpu.sync_copy(zeros_hbm.at[pl.ds(rbase, RPT)],
                    acc_sh.at[pl.ds(rbase, RPT)])
    pltpu.sync_copy(ones_hbm, ones_v)
    pltpu.sync_copy(dst_hbm.at[pl.ds(wid * CPT, CPT)], idxd_v)
    plsc.subcore_barrier()

    def scat(c, b):
        pltpu.async_copy(ones_v, acc_sh.at[idxd_v.at[c]], ssem[b], add=True)

    def swait(c, b):
        pltpu.make_async_copy(ones_v, acc_sh.at[idxd_v.at[c]], ssem[b]).wait()

    def group(j0, carry):
        j = j0 * NB
        for b in range(NB):
            scat(j + b, b)
        for b in range(NB):
            swait(j + b, b)
        return carry

    lax.fori_loop(0, NGRP, group, 0)

    plsc.subcore_barrier()
    pltpu.sync_copy(acc_sh.at[pl.ds(rbase, RPT)],
                    out_hbm.at[cid, pl.ds(rbase, RPT)])


@functools.partial(
    pl.kernel,
    out_type=jax.ShapeDtypeStruct((NC, N, D), jnp.float32),
    mesh=_mesh,
    scratch_types=[
        pltpu.VMEM((CPT, CHR), jnp.int32),
        pltpu.VMEM((CPT, CHR), jnp.int32),
        pltpu.VMEM((NB, CHR, D), jnp.float32),
        pltpu.VMEM_SHARED((N, D), jnp.float32),
        pltpu.SemaphoreType.DMA,
        pltpu.SemaphoreType.DMA,
        pltpu.SemaphoreType.DMA,
        pltpu.SemaphoreType.DMA,
        pltpu.SemaphoreType.DMA,
        pltpu.SemaphoreType.DMA,
        pltpu.SemaphoreType.DMA,
        pltpu.SemaphoreType.DMA,
    ],
    compiler_params=_sc_params,
)
def _sc_edge_sum(y_hbm, src_hbm, dst_hbm, zeros_hbm, out_hbm,
                 idxs_v, idxd_v, rows_v, acc_sh,
                 g0, g1, g2, g3, s0, s1, s2, s3):
    gsem = (g0, g1, g2, g3)
    ssem = (s0, s1, s2, s3)
    cid = lax.axis_index("c")
    sid = lax.axis_index("s")
    wid = sid * NC + cid
    rbase = sid * RPT
    pltpu.sync_copy(zeros_hbm.at[pl.ds(rbase, RPT)],
                    acc_sh.at[pl.ds(rbase, RPT)])
    cbase = wid * CPT
    pltpu.sync_copy(src_hbm.at[pl.ds(cbase, CPT)], idxs_v)
    pltpu.sync_copy(dst_hbm.at[pl.ds(cbase, CPT)], idxd_v)
    plsc.subcore_barrier()

    def gath(c, b):
        pltpu.async_copy(y_hbm.at[idxs_v.at[c]], rows_v.at[b], gsem[b])

    def gwait(c, b):
        pltpu.make_async_copy(y_hbm.at[idxs_v.at[c]], rows_v.at[b],
                              gsem[b]).wait()

    def scat(c, b):
        pltpu.async_copy(rows_v.at[b], acc_sh.at[idxd_v.at[c]], ssem[b],
                         add=True)

    def swait(c, b):
        pltpu.make_async_copy(rows_v.at[b], acc_sh.at[idxd_v.at[c]],
                              ssem[b]).wait()

    for b in range(NB):
        gath(b, b)

    def group(j0, carry):
        j = j0 * NB
        for b in range(NB):
            gwait(j + b, b)
            scat(j + b, b)
        for b in range(NB):
            swait(j + b, b)
            gath(j + NB + b, b)
        return carry

    lax.fori_loop(0, NGRP - 1, group, 0)
    j = (NGRP - 1) * NB
    for b in range(NB):
        gwait(j + b, b)
        scat(j + b, b)
    for b in range(NB):
        swait(j + b, b)

    plsc.subcore_barrier()
    pltpu.sync_copy(acc_sh.at[pl.ds(rbase, RPT)],
                    out_hbm.at[cid, pl.ds(rbase, RPT)])


def _dvec(degp_ref):
    deg = 1.0 + degp_ref[0, :, :1] + degp_ref[1, :, :1]
    return lax.rsqrt(deg)


def _tc1_body(degp_ref, x_ref, w1_ref, y1_ref):
    d = _dvec(degp_ref)
    xw = jnp.dot(x_ref[...], w1_ref[...],
                 preferred_element_type=jnp.float32,
                 precision=lax.Precision.HIGHEST)
    y1_ref[...] = xw * d


def _tc2_body(degp_ref, s_ref, y1_ref, w2_ref, b1_ref, y2_ref):
    d = _dvec(degp_ref)
    s = s_ref[0] + s_ref[1] + y1_ref[...]
    h = jnp.maximum(d * s + b1_ref[...], 0.0)
    hw = jnp.dot(h, w2_ref[...],
                 preferred_element_type=jnp.float32,
                 precision=lax.Precision.HIGHEST)
    y2_ref[...] = hw * d


def _tc3_body(degp_ref, s_ref, y2_ref, b2_ref, batch_ref, out_ref, acc, cnt):
    i = pl.program_id(0)

    @pl.when(i == 0)
    def _init():
        acc[...] = jnp.zeros_like(acc)
        cnt[...] = jnp.zeros_like(cnt)

    d = _dvec(degp_ref)
    o = d * (s_ref[0] + s_ref[1] + y2_ref[...]) + b2_ref[...]
    seg = batch_ref[0, 0, :]
    oh = (lax.broadcasted_iota(jnp.int32, (G, BN), 0)
          == seg[None, :]).astype(jnp.float32)
    acc[...] += jnp.dot(oh, o, preferred_element_type=jnp.float32,
                        precision=lax.Precision.HIGHEST)
    cnt[...] += jnp.sum(oh, axis=1, keepdims=True)

    @pl.when(i == NBLK - 1)
    def _fin():
        out_ref[...] = acc[...] / jnp.maximum(cnt[...], 1.0)


_row = lambda i: (i, 0)
_fix2 = lambda i: (0, 0)

_tc1 = pl.pallas_call(
    _tc1_body,
    grid=(NBLK,),
    in_specs=[
        pl.BlockSpec((NC, BN, 16), lambda i: (0, i, 0)),
        pl.BlockSpec((BN, D), _row),
        pl.BlockSpec((D, D), _fix2),
    ],
    out_specs=pl.BlockSpec((BN, D), _row),
    out_shape=jax.ShapeDtypeStruct((N, D), jnp.float32),
)

_tc2 = pl.pallas_call(
    _tc2_body,
    grid=(NBLK,),
    in_specs=[
        pl.BlockSpec((NC, BN, 16), lambda i: (0, i, 0)),
        pl.BlockSpec((NC, BN, D), lambda i: (0, i, 0)),
        pl.BlockSpec((BN, D), _row),
        pl.BlockSpec((D, D), _fix2),
        pl.BlockSpec((1, D), _fix2),
    ],
    out_specs=pl.BlockSpec((BN, D), _row),
    out_shape=jax.ShapeDtypeStruct((N, D), jnp.float32),
)

_tc3 = pl.pallas_call(
    _tc3_body,
    grid=(NBLK,),
    in_specs=[
        pl.BlockSpec((NC, BN, 16), lambda i: (0, i, 0)),
        pl.BlockSpec((NC, BN, D), lambda i: (0, i, 0)),
        pl.BlockSpec((BN, D), _row),
        pl.BlockSpec((1, D), _fix2),
        pl.BlockSpec((1, 1, BN), lambda i: (i, 0, 0)),
    ],
    out_specs=pl.BlockSpec((G, D), _fix2),
    out_shape=jax.ShapeDtypeStruct((G, D), jnp.float32),
    scratch_shapes=[
        pltpu.VMEM((G, D), jnp.float32),
        pltpu.VMEM((G, 1), jnp.float32),
    ],
)


def kernel(x, edge_index, batch, W1, b1, W2, b2):
    x = x.astype(jnp.float32)
    src2d = edge_index[0].reshape(ROWS, CHR)
    dst2d = edge_index[1].reshape(ROWS, CHR)
    ones16 = jnp.ones((CHR, 16), jnp.float32)
    zeros16 = jnp.zeros((N, 16), jnp.float32)
    zerosND = jnp.zeros((N, D), jnp.float32)
    b1r = b1.reshape(1, D)
    b2r = b2.reshape(1, D)
    batch3 = batch.reshape(NBLK, 1, BN)

    degp = _sc_degree(dst2d, ones16, zeros16)
    y1 = _tc1(degp, x, W1)
    s1 = _sc_edge_sum(y1, src2d, dst2d, zerosND)
    y2 = _tc2(degp, s1, y1, W2, b1r)
    s2 = _sc_edge_sum(y2, src2d, dst2d, zerosND)
    out = _tc3(degp, s2, y2, b2r, batch3)
    return out
